# baseline (speedup 1.0000x reference)
from functools import partial

import jax
import jax.numpy as jnp
from jax import lax
from jax.experimental import pallas as pl
from jax.experimental.pallas import tpu as pltpu

EPS = 1e-5


def _phase_split_kernel(x_ref, sel_ref, p_ref, *, cin, ho, wo):
    w = 2 * wo
    v = x_ref[0]
    e = v[:, :, :w].reshape(cin * ho, w).astype(jnp.bfloat16)
    o = v[:, :, w:].reshape(cin * ho, w).astype(jnp.bfloat16)
    re = jnp.dot(e, sel_ref[...], preferred_element_type=jnp.float32)
    ro = jnp.dot(o, sel_ref[...], preferred_element_type=jnp.float32)
    a = re[:, :wo].reshape(cin, ho, wo)
    b = re[:, wo:].reshape(cin, ho, wo)
    c = ro[:, :wo].reshape(cin, ho, wo)
    d = ro[:, wo:].reshape(cin, ho, wo)
    p_ref[0] = jnp.concatenate([a, b, c, d], axis=0).astype(jnp.bfloat16)


def _conv_stats_kernel(p_ref, w1t_ref, y1_ref, s1_ref, q1_ref, *, cin, wo, hwo):
    p = p_ref[0]
    a = p[:cin]
    b = p[cin:2 * cin]
    c = p[2 * cin:3 * cin]
    d = p[3 * cin:]

    col = lax.broadcasted_iota(jnp.int32, (1, hwo), 1)
    col_ok = (col % wo != 0).astype(p.dtype)

    def rshift(t, amt):
        return jnp.concatenate(
            [jnp.zeros((cin, amt), t.dtype), t[:, : hwo - amt]], axis=1)

    d_rc = rshift(d, wo + 1) * col_ok
    c_r = rshift(c, wo)
    d_r = rshift(d, wo)
    b_c = rshift(b, 1) * col_ok
    d_c = rshift(d, 1) * col_ok

    taps = jnp.concatenate([d_rc, c_r, d_r, b_c, a, b, d_c, c, d], axis=0)
    y1 = jnp.dot(w1t_ref[...], taps, preferred_element_type=jnp.float32)
    y1_ref[0] = y1
    s1_ref[0] = jnp.sum(y1, axis=1, keepdims=True)
    q1_ref[0] = jnp.sum(y1 * y1, axis=1, keepdims=True)


def _bn_conv1x1_stats_kernel(y1_ref, sc1_ref, sh1_ref, w2t_ref, y2_ref, s2_ref, q2_ref):
    z = jnp.maximum(y1_ref[0] * sc1_ref[...] + sh1_ref[...], 0.0)
    y2 = jnp.dot(w2t_ref[...], z.astype(w2t_ref.dtype),
                 preferred_element_type=jnp.float32)
    y2_ref[0] = y2
    s2_ref[0] = jnp.sum(y2, axis=1, keepdims=True)
    q2_ref[0] = jnp.sum(y2 * y2, axis=1, keepdims=True)


def _bn_relu_kernel(y2_ref, sc2_ref, sh2_ref, out_ref):
    out_ref[0] = jnp.maximum(y2_ref[0] * sc2_ref[...] + sh2_ref[...], 0.0)


def kernel(x, w1, w2, g1, b1, g2, b2):
    n, cin, h, w = x.shape
    kh, kw, _, cout = w1.shape
    ho, wo = h // 2, w // 2
    hwo = ho * wo
    m = n * hwo
    k = kh * kw * cin

    x8 = x.reshape(n, cin, ho, 2 * w)
    i_in = lax.broadcasted_iota(jnp.int32, (w, w), 0)
    j_out = lax.broadcasted_iota(jnp.int32, (w, w), 1)
    sel = ((i_in == 2 * j_out) | (i_in == 2 * (j_out - wo) + 1)).astype(jnp.bfloat16)
    w1t = jnp.transpose(w1, (3, 0, 1, 2)).reshape(cout, k).astype(jnp.bfloat16)
    w2t = w2.T.astype(jnp.bfloat16)

    stat_shape = jax.ShapeDtypeStruct((n, cout, 1), jnp.float32)
    stat_spec = pl.BlockSpec((1, cout, 1), lambda i: (i, 0, 0))
    plane_spec = pl.BlockSpec((1, cout, hwo), lambda i: (i, 0, 0))
    vec_spec = pl.BlockSpec((cout, 1), lambda i: (0, 0))
    parallel = pltpu.CompilerParams(dimension_semantics=("parallel",))
    inv_m = 1.0 / float(m)

    phases = pl.pallas_call(
        partial(_phase_split_kernel, cin=cin, ho=ho, wo=wo),
        grid=(n,),
        in_specs=[pl.BlockSpec((1, cin, ho, 2 * w), lambda i: (i, 0, 0, 0)),
                  pl.BlockSpec((w, w), lambda i: (0, 0))],
        out_specs=pl.BlockSpec((1, 4 * cin, ho, wo), lambda i: (i, 0, 0, 0)),
        out_shape=jax.ShapeDtypeStruct((n, 4 * cin, ho, wo), jnp.bfloat16),
        compiler_params=parallel,
    )(x8, sel)
    phases = phases.reshape(n, 4 * cin, hwo)

    y1, s1, q1 = pl.pallas_call(
        partial(_conv_stats_kernel, cin=cin, wo=wo, hwo=hwo),
        grid=(n,),
        in_specs=[pl.BlockSpec((1, 4 * cin, hwo), lambda i: (i, 0, 0)),
                  pl.BlockSpec((cout, k), lambda i: (0, 0))],
        out_specs=(plane_spec, stat_spec, stat_spec),
        out_shape=(jax.ShapeDtypeStruct((n, cout, hwo), jnp.float32),
                   stat_shape, stat_shape),
        compiler_params=parallel,
    )(phases, w1t)

    mean1 = jnp.sum(s1, axis=0) * inv_m
    var1 = jnp.sum(q1, axis=0) * inv_m - mean1 * mean1
    sc1 = g1.reshape(cout, 1) * lax.rsqrt(var1 + EPS)
    sh1 = b1.reshape(cout, 1) - mean1 * sc1

    y2, s2, q2 = pl.pallas_call(
        _bn_conv1x1_stats_kernel,
        grid=(n,),
        in_specs=[plane_spec, vec_spec, vec_spec,
                  pl.BlockSpec((cout, cout), lambda i: (0, 0))],
        out_specs=(plane_spec, stat_spec, stat_spec),
        out_shape=(jax.ShapeDtypeStruct((n, cout, hwo), jnp.float32),
                   stat_shape, stat_shape),
        compiler_params=parallel,
    )(y1, sc1, sh1, w2t)

    mean2 = jnp.sum(s2, axis=0) * inv_m
    var2 = jnp.sum(q2, axis=0) * inv_m - mean2 * mean2
    sc2 = g2.reshape(cout, 1) * lax.rsqrt(var2 + EPS)
    sh2 = b2.reshape(cout, 1) - mean2 * sc2

    out = pl.pallas_call(
        _bn_relu_kernel,
        grid=(n,),
        in_specs=[plane_spec, vec_spec, vec_spec],
        out_specs=plane_spec,
        out_shape=jax.ShapeDtypeStruct((n, cout, hwo), jnp.float32),
        compiler_params=parallel,
    )(y2, sc2, sh2)

    return out.reshape(n, cout, ho, wo)


# speedup vs baseline: 1.4244x; 1.4244x over previous
from functools import partial

import jax
import jax.numpy as jnp
from jax import lax
from jax.experimental import pallas as pl
from jax.experimental.pallas import tpu as pltpu

EPS = 1e-5


def _phase_split_kernel(x_ref, sel_ref, p_ref, *, cin, ho, wo):
    w = 2 * wo
    v = x_ref[0]
    e = v[:, :, :w].reshape(cin * ho, w).astype(jnp.bfloat16)
    o = v[:, :, w:].reshape(cin * ho, w).astype(jnp.bfloat16)
    re = jnp.dot(e, sel_ref[...], preferred_element_type=jnp.float32)
    ro = jnp.dot(o, sel_ref[...], preferred_element_type=jnp.float32)
    re = re.astype(jnp.bfloat16).reshape(cin, ho, w)
    ro = ro.astype(jnp.bfloat16).reshape(cin, ho, w)
    for r in range(ho):
        seg = pl.ds(r * wo, wo)
        p_ref[0, :cin, seg] = re[:, r, :wo]
        p_ref[0, cin:2 * cin, seg] = re[:, r, wo:]
        p_ref[0, 2 * cin:3 * cin, seg] = ro[:, r, :wo]
        p_ref[0, 3 * cin:, seg] = ro[:, r, wo:]


def _conv_stats_kernel(p_ref, w1t_ref, y1_ref, s1_ref, q1_ref, *, cin, wo, hwo):
    p = p_ref[0]
    a = p[:cin]
    b = p[cin:2 * cin]
    c = p[2 * cin:3 * cin]
    d = p[3 * cin:]

    col = lax.broadcasted_iota(jnp.int32, (1, hwo), 1)
    col_ok = (col % wo != 0).astype(p.dtype)

    def rshift(t, amt):
        return jnp.concatenate(
            [jnp.zeros((cin, amt), t.dtype), t[:, : hwo - amt]], axis=1)

    d_rc = rshift(d, wo + 1) * col_ok
    c_r = rshift(c, wo)
    d_r = rshift(d, wo)
    b_c = rshift(b, 1) * col_ok
    d_c = rshift(d, 1) * col_ok

    taps = jnp.concatenate([d_rc, c_r, d_r, b_c, a, b, d_c, c, d], axis=0)
    y1 = jnp.dot(w1t_ref[...], taps, preferred_element_type=jnp.float32)
    y1_ref[0] = y1
    s1_ref[0] = jnp.sum(y1, axis=1, keepdims=True)
    q1_ref[0] = jnp.sum(y1 * y1, axis=1, keepdims=True)


def _bn_conv1x1_stats_kernel(y1_ref, sc1_ref, sh1_ref, w2t_ref, y2_ref, s2_ref, q2_ref):
    z = jnp.maximum(y1_ref[0] * sc1_ref[...] + sh1_ref[...], 0.0)
    y2 = jnp.dot(w2t_ref[...], z.astype(w2t_ref.dtype),
                 preferred_element_type=jnp.float32)
    y2_ref[0] = y2
    s2_ref[0] = jnp.sum(y2, axis=1, keepdims=True)
    q2_ref[0] = jnp.sum(y2 * y2, axis=1, keepdims=True)


def _bn_relu_kernel(y2_ref, sc2_ref, sh2_ref, out_ref):
    out_ref[0] = jnp.maximum(y2_ref[0] * sc2_ref[...] + sh2_ref[...], 0.0)


def kernel(x, w1, w2, g1, b1, g2, b2):
    n, cin, h, w = x.shape
    kh, kw, _, cout = w1.shape
    ho, wo = h // 2, w // 2
    hwo = ho * wo
    m = n * hwo
    k = kh * kw * cin

    x8 = x.reshape(n, cin, ho, 2 * w)
    i_in = lax.broadcasted_iota(jnp.int32, (w, w), 0)
    j_out = lax.broadcasted_iota(jnp.int32, (w, w), 1)
    sel = ((i_in == 2 * j_out) | (i_in == 2 * (j_out - wo) + 1)).astype(jnp.bfloat16)
    w1t = jnp.transpose(w1, (3, 0, 1, 2)).reshape(cout, k).astype(jnp.bfloat16)
    w2t = w2.T.astype(jnp.bfloat16)

    stat_shape = jax.ShapeDtypeStruct((n, cout, 1), jnp.float32)
    stat_spec = pl.BlockSpec((1, cout, 1), lambda i: (i, 0, 0))
    plane_spec = pl.BlockSpec((1, cout, hwo), lambda i: (i, 0, 0))
    vec_spec = pl.BlockSpec((cout, 1), lambda i: (0, 0))
    parallel = pltpu.CompilerParams(dimension_semantics=("parallel",))
    inv_m = 1.0 / float(m)

    phases = pl.pallas_call(
        partial(_phase_split_kernel, cin=cin, ho=ho, wo=wo),
        grid=(n,),
        in_specs=[pl.BlockSpec((1, cin, ho, 2 * w), lambda i: (i, 0, 0, 0)),
                  pl.BlockSpec((w, w), lambda i: (0, 0))],
        out_specs=pl.BlockSpec((1, 4 * cin, hwo), lambda i: (i, 0, 0)),
        out_shape=jax.ShapeDtypeStruct((n, 4 * cin, hwo), jnp.bfloat16),
        compiler_params=parallel,
    )(x8, sel)

    y1, s1, q1 = pl.pallas_call(
        partial(_conv_stats_kernel, cin=cin, wo=wo, hwo=hwo),
        grid=(n,),
        in_specs=[pl.BlockSpec((1, 4 * cin, hwo), lambda i: (i, 0, 0)),
                  pl.BlockSpec((cout, k), lambda i: (0, 0))],
        out_specs=(plane_spec, stat_spec, stat_spec),
        out_shape=(jax.ShapeDtypeStruct((n, cout, hwo), jnp.float32),
                   stat_shape, stat_shape),
        compiler_params=parallel,
    )(phases, w1t)

    mean1 = jnp.sum(s1, axis=0) * inv_m
    var1 = jnp.sum(q1, axis=0) * inv_m - mean1 * mean1
    sc1 = g1.reshape(cout, 1) * lax.rsqrt(var1 + EPS)
    sh1 = b1.reshape(cout, 1) - mean1 * sc1

    y2, s2, q2 = pl.pallas_call(
        _bn_conv1x1_stats_kernel,
        grid=(n,),
        in_specs=[plane_spec, vec_spec, vec_spec,
                  pl.BlockSpec((cout, cout), lambda i: (0, 0))],
        out_specs=(plane_spec, stat_spec, stat_spec),
        out_shape=(jax.ShapeDtypeStruct((n, cout, hwo), jnp.float32),
                   stat_shape, stat_shape),
        compiler_params=parallel,
    )(y1, sc1, sh1, w2t)

    mean2 = jnp.sum(s2, axis=0) * inv_m
    var2 = jnp.sum(q2, axis=0) * inv_m - mean2 * mean2
    sc2 = g2.reshape(cout, 1) * lax.rsqrt(var2 + EPS)
    sh2 = b2.reshape(cout, 1) - mean2 * sc2

    out = pl.pallas_call(
        _bn_relu_kernel,
        grid=(n,),
        in_specs=[plane_spec, vec_spec, vec_spec],
        out_specs=plane_spec,
        out_shape=jax.ShapeDtypeStruct((n, cout, hwo), jnp.float32),
        compiler_params=parallel,
    )(y2, sc2, sh2)

    return out.reshape(n, cout, ho, wo)


# speedup vs baseline: 1.4809x; 1.0397x over previous
from functools import partial

import jax
import jax.numpy as jnp
from jax import lax
from jax.experimental import pallas as pl
from jax.experimental.pallas import tpu as pltpu

EPS = 1e-5


def _phase_split_kernel(x_ref, sel_ref, p_ref, *, cin, ho, wo):
    w = 2 * wo
    v = x_ref[0]
    e = v[:, :, :w].reshape(cin * ho, w).astype(jnp.bfloat16)
    o = v[:, :, w:].reshape(cin * ho, w).astype(jnp.bfloat16)
    re = jnp.dot(e, sel_ref[...], preferred_element_type=jnp.float32)
    ro = jnp.dot(o, sel_ref[...], preferred_element_type=jnp.float32)
    re = re.astype(jnp.bfloat16).reshape(cin, ho, w)
    ro = ro.astype(jnp.bfloat16).reshape(cin, ho, w)
    for r in range(ho):
        seg = pl.ds(r * wo, wo)
        p_ref[0, :cin, seg] = re[:, r, :wo]
        p_ref[0, cin:2 * cin, seg] = re[:, r, wo:]
        p_ref[0, 2 * cin:3 * cin, seg] = ro[:, r, :wo]
        p_ref[0, 3 * cin:, seg] = ro[:, r, wo:]


def _conv_stats_kernel(p_ref, w1t_ref, y1_ref, s1_ref, q1_ref, *, cin, wo, hwo):
    p = p_ref[0]
    a = p[:cin]
    b = p[cin:2 * cin]
    c = p[2 * cin:3 * cin]
    d = p[3 * cin:]

    col = lax.broadcasted_iota(jnp.int32, (1, hwo), 1)
    col_ok = (col % wo != 0).astype(p.dtype)

    def rshift(t, amt):
        return jnp.concatenate(
            [jnp.zeros((cin, amt), t.dtype), t[:, : hwo - amt]], axis=1)

    d_rc = rshift(d, wo + 1) * col_ok
    c_r = rshift(c, wo)
    d_r = rshift(d, wo)
    b_c = rshift(b, 1) * col_ok
    d_c = rshift(d, 1) * col_ok

    taps = jnp.concatenate([d_rc, c_r, d_r, b_c, a, b, d_c, c, d], axis=0)
    y1 = jnp.dot(w1t_ref[...], taps, preferred_element_type=jnp.float32)
    y1_ref[0] = y1.astype(jnp.bfloat16)
    s1_ref[0] = jnp.sum(y1, axis=1, keepdims=True)
    q1_ref[0] = jnp.sum(y1 * y1, axis=1, keepdims=True)


def _bn_conv1x1_stats_kernel(y1_ref, sc1_ref, sh1_ref, w2t_ref, y2_ref, s2_ref, q2_ref):
    z = jnp.maximum(y1_ref[0].astype(jnp.float32) * sc1_ref[...] + sh1_ref[...], 0.0)
    y2 = jnp.dot(w2t_ref[...], z.astype(w2t_ref.dtype),
                 preferred_element_type=jnp.float32)
    y2_ref[0] = y2.astype(jnp.bfloat16)
    s2_ref[0] = jnp.sum(y2, axis=1, keepdims=True)
    q2_ref[0] = jnp.sum(y2 * y2, axis=1, keepdims=True)


def _bn_relu_kernel(y2_ref, sc2_ref, sh2_ref, out_ref):
    out_ref[0] = jnp.maximum(
        y2_ref[0].astype(jnp.float32) * sc2_ref[...] + sh2_ref[...], 0.0)


def kernel(x, w1, w2, g1, b1, g2, b2):
    n, cin, h, w = x.shape
    kh, kw, _, cout = w1.shape
    ho, wo = h // 2, w // 2
    hwo = ho * wo
    m = n * hwo
    k = kh * kw * cin

    x8 = x.reshape(n, cin, ho, 2 * w)
    i_in = lax.broadcasted_iota(jnp.int32, (w, w), 0)
    j_out = lax.broadcasted_iota(jnp.int32, (w, w), 1)
    sel = ((i_in == 2 * j_out) | (i_in == 2 * (j_out - wo) + 1)).astype(jnp.bfloat16)
    w1t = jnp.transpose(w1, (3, 0, 1, 2)).reshape(cout, k).astype(jnp.bfloat16)
    w2t = w2.T.astype(jnp.bfloat16)

    stat_shape = jax.ShapeDtypeStruct((n, cout, 1), jnp.float32)
    stat_spec = pl.BlockSpec((1, cout, 1), lambda i: (i, 0, 0))
    plane_spec = pl.BlockSpec((1, cout, hwo), lambda i: (i, 0, 0))
    vec_spec = pl.BlockSpec((cout, 1), lambda i: (0, 0))
    parallel = pltpu.CompilerParams(dimension_semantics=("parallel",))
    inv_m = 1.0 / float(m)

    phases = pl.pallas_call(
        partial(_phase_split_kernel, cin=cin, ho=ho, wo=wo),
        grid=(n,),
        in_specs=[pl.BlockSpec((1, cin, ho, 2 * w), lambda i: (i, 0, 0, 0)),
                  pl.BlockSpec((w, w), lambda i: (0, 0))],
        out_specs=pl.BlockSpec((1, 4 * cin, hwo), lambda i: (i, 0, 0)),
        out_shape=jax.ShapeDtypeStruct((n, 4 * cin, hwo), jnp.bfloat16),
        compiler_params=parallel,
    )(x8, sel)

    y1, s1, q1 = pl.pallas_call(
        partial(_conv_stats_kernel, cin=cin, wo=wo, hwo=hwo),
        grid=(n,),
        in_specs=[pl.BlockSpec((1, 4 * cin, hwo), lambda i: (i, 0, 0)),
                  pl.BlockSpec((cout, k), lambda i: (0, 0))],
        out_specs=(plane_spec, stat_spec, stat_spec),
        out_shape=(jax.ShapeDtypeStruct((n, cout, hwo), jnp.bfloat16),
                   stat_shape, stat_shape),
        compiler_params=parallel,
    )(phases, w1t)

    mean1 = jnp.sum(s1, axis=0) * inv_m
    var1 = jnp.sum(q1, axis=0) * inv_m - mean1 * mean1
    sc1 = g1.reshape(cout, 1) * lax.rsqrt(var1 + EPS)
    sh1 = b1.reshape(cout, 1) - mean1 * sc1

    y2, s2, q2 = pl.pallas_call(
        _bn_conv1x1_stats_kernel,
        grid=(n,),
        in_specs=[plane_spec, vec_spec, vec_spec,
                  pl.BlockSpec((cout, cout), lambda i: (0, 0))],
        out_specs=(plane_spec, stat_spec, stat_spec),
        out_shape=(jax.ShapeDtypeStruct((n, cout, hwo), jnp.bfloat16),
                   stat_shape, stat_shape),
        compiler_params=parallel,
    )(y1, sc1, sh1, w2t)

    mean2 = jnp.sum(s2, axis=0) * inv_m
    var2 = jnp.sum(q2, axis=0) * inv_m - mean2 * mean2
    sc2 = g2.reshape(cout, 1) * lax.rsqrt(var2 + EPS)
    sh2 = b2.reshape(cout, 1) - mean2 * sc2

    out = pl.pallas_call(
        _bn_relu_kernel,
        grid=(n,),
        in_specs=[plane_spec, vec_spec, vec_spec],
        out_specs=plane_spec,
        out_shape=jax.ShapeDtypeStruct((n, cout, hwo), jnp.float32),
        compiler_params=parallel,
    )(y2, sc2, sh2)

    return out.reshape(n, cout, ho, wo)


# speedup vs baseline: 2.1096x; 1.4245x over previous
from functools import partial

import jax
import jax.numpy as jnp
from jax import lax
from jax.experimental import pallas as pl
from jax.experimental.pallas import tpu as pltpu

EPS = 1e-5


def _phase_split_kernel(x_ref, sel_ref, p_ref, *, cin, ho, wo):
    v = x_ref[0]
    for r in range(ho):
        pair = v[:, 2 * r, :].astype(jnp.bfloat16)
        odd = v[:, 2 * r + 1, :].astype(jnp.bfloat16)
        both = jnp.concatenate([pair, odd], axis=0)
        res = jnp.dot(both, sel_ref[...],
                      preferred_element_type=jnp.float32).astype(jnp.bfloat16)
        seg = pl.ds(r * wo, wo)
        p_ref[0, :cin, seg] = res[:cin, :wo]
        p_ref[0, cin:2 * cin, seg] = res[:cin, wo:]
        p_ref[0, 2 * cin:3 * cin, seg] = res[cin:, :wo]
        p_ref[0, 3 * cin:, seg] = res[cin:, wo:]


def _conv_stats_kernel(p_ref, w1t_ref, y1_ref, s1_ref, q1_ref, *, cin, wo, hwo):
    p = p_ref[0]
    a = p[:cin]
    b = p[cin:2 * cin]
    c = p[2 * cin:3 * cin]
    d = p[3 * cin:]

    col = lax.broadcasted_iota(jnp.int32, (1, hwo), 1)
    col_ok = (col % wo != 0).astype(p.dtype)

    def rshift(t, amt):
        return jnp.concatenate(
            [jnp.zeros((cin, amt), t.dtype), t[:, : hwo - amt]], axis=1)

    d_rc = rshift(d, wo + 1) * col_ok
    c_r = rshift(c, wo)
    d_r = rshift(d, wo)
    b_c = rshift(b, 1) * col_ok
    d_c = rshift(d, 1) * col_ok

    taps = jnp.concatenate([d_rc, c_r, d_r, b_c, a, b, d_c, c, d], axis=0)
    y1 = jnp.dot(w1t_ref[...], taps, preferred_element_type=jnp.float32)
    y1_ref[0] = y1.astype(jnp.bfloat16)
    s1_ref[0] = jnp.sum(y1, axis=1, keepdims=True)
    q1_ref[0] = jnp.sum(y1 * y1, axis=1, keepdims=True)


def _bn_conv1x1_stats_kernel(y1_ref, sc1_ref, sh1_ref, w2t_ref, y2_ref, s2_ref, q2_ref):
    z = jnp.maximum(y1_ref[0].astype(jnp.float32) * sc1_ref[...] + sh1_ref[...], 0.0)
    y2 = jnp.dot(w2t_ref[...], z.astype(w2t_ref.dtype),
                 preferred_element_type=jnp.float32)
    y2_ref[0] = y2.astype(jnp.bfloat16)
    s2_ref[0] = jnp.sum(y2, axis=1, keepdims=True)
    q2_ref[0] = jnp.sum(y2 * y2, axis=1, keepdims=True)


def _bn_relu_kernel(y2_ref, sc2_ref, sh2_ref, out_ref):
    out_ref[0] = jnp.maximum(
        y2_ref[0].astype(jnp.float32) * sc2_ref[...] + sh2_ref[...], 0.0)


def kernel(x, w1, w2, g1, b1, g2, b2):
    n, cin, h, w = x.shape
    kh, kw, _, cout = w1.shape
    ho, wo = h // 2, w // 2
    hwo = ho * wo
    m = n * hwo
    k = kh * kw * cin

    i_in = lax.broadcasted_iota(jnp.int32, (w, w), 0)
    j_out = lax.broadcasted_iota(jnp.int32, (w, w), 1)
    sel = ((i_in == 2 * j_out) | (i_in == 2 * (j_out - wo) + 1)).astype(jnp.bfloat16)
    w1t = jnp.transpose(w1, (3, 0, 1, 2)).reshape(cout, k).astype(jnp.bfloat16)
    w2t = w2.T.astype(jnp.bfloat16)

    stat_shape = jax.ShapeDtypeStruct((n, cout, 1), jnp.float32)
    stat_spec = pl.BlockSpec((1, cout, 1), lambda i: (i, 0, 0))
    plane_spec = pl.BlockSpec((1, cout, hwo), lambda i: (i, 0, 0))
    vec_spec = pl.BlockSpec((cout, 1), lambda i: (0, 0))
    parallel = pltpu.CompilerParams(dimension_semantics=("parallel",))
    inv_m = 1.0 / float(m)

    phases = pl.pallas_call(
        partial(_phase_split_kernel, cin=cin, ho=ho, wo=wo),
        grid=(n,),
        in_specs=[pl.BlockSpec((1, cin, h, w), lambda i: (i, 0, 0, 0)),
                  pl.BlockSpec((w, w), lambda i: (0, 0))],
        out_specs=pl.BlockSpec((1, 4 * cin, hwo), lambda i: (i, 0, 0)),
        out_shape=jax.ShapeDtypeStruct((n, 4 * cin, hwo), jnp.bfloat16),
        compiler_params=parallel,
    )(x, sel)

    y1, s1, q1 = pl.pallas_call(
        partial(_conv_stats_kernel, cin=cin, wo=wo, hwo=hwo),
        grid=(n,),
        in_specs=[pl.BlockSpec((1, 4 * cin, hwo), lambda i: (i, 0, 0)),
                  pl.BlockSpec((cout, k), lambda i: (0, 0))],
        out_specs=(plane_spec, stat_spec, stat_spec),
        out_shape=(jax.ShapeDtypeStruct((n, cout, hwo), jnp.bfloat16),
                   stat_shape, stat_shape),
        compiler_params=parallel,
    )(phases, w1t)

    mean1 = jnp.sum(s1, axis=0) * inv_m
    var1 = jnp.sum(q1, axis=0) * inv_m - mean1 * mean1
    sc1 = g1.reshape(cout, 1) * lax.rsqrt(var1 + EPS)
    sh1 = b1.reshape(cout, 1) - mean1 * sc1

    y2, s2, q2 = pl.pallas_call(
        _bn_conv1x1_stats_kernel,
        grid=(n,),
        in_specs=[plane_spec, vec_spec, vec_spec,
                  pl.BlockSpec((cout, cout), lambda i: (0, 0))],
        out_specs=(plane_spec, stat_spec, stat_spec),
        out_shape=(jax.ShapeDtypeStruct((n, cout, hwo), jnp.bfloat16),
                   stat_shape, stat_shape),
        compiler_params=parallel,
    )(y1, sc1, sh1, w2t)

    mean2 = jnp.sum(s2, axis=0) * inv_m
    var2 = jnp.sum(q2, axis=0) * inv_m - mean2 * mean2
    sc2 = g2.reshape(cout, 1) * lax.rsqrt(var2 + EPS)
    sh2 = b2.reshape(cout, 1) - mean2 * sc2

    out = pl.pallas_call(
        _bn_relu_kernel,
        grid=(n,),
        in_specs=[plane_spec, vec_spec, vec_spec],
        out_specs=plane_spec,
        out_shape=jax.ShapeDtypeStruct((n, cout, hwo), jnp.float32),
        compiler_params=parallel,
    )(y2, sc2, sh2)

    return out.reshape(n, cout, ho, wo)
